# baseline (speedup 1.0000x reference)
import jax
import jax.numpy as jnp
import numpy as np
from jax.experimental import pallas as pl
from jax.experimental.pallas import tpu as pltpu
from jax.sharding import PartitionSpec as P

DIM_STATE = 32
N_PARTICLES = 1024
DIM_OBS = 32
HIDDEN = 64
BATCH = 64
SEQ = 16

NDEV = 2 if jax.device_count() >= 2 else 1
B_LOC = BATCH // NDEV
ROWS_L = B_LOC * N_PARTICLES

_TINY = np.float32(np.finfo(np.float32).tiny)
_LO = np.float32(np.nextafter(np.float32(-1.0), np.float32(0.0)))

PBLK = 128
PCHUNK = 4
NCHUNK = PBLK // PCHUNK
NPBLK = N_PARTICLES // PBLK
KLOOP = max(NCHUNK, B_LOC)


def _threefry_xor(kd0, kd1, x1):
    return _threefry_core(kd0, kd1, x1 + kd1)


def _threefry_core(ks0, ks1, x1):
    ks2 = ks0 ^ ks1 ^ jnp.uint32(0x1BD11BDA)
    x0 = jnp.zeros_like(x1) + ks0
    rots = ((13, 15, 26, 6), (17, 29, 16, 24))
    ks = (ks0, ks1, ks2)

    def rotl(x, d):
        return (x << jnp.uint32(d)) | (x >> jnp.uint32(32 - d))

    for i in range(5):
        for r in rots[i % 2]:
            x0 = x0 + x1
            x1 = rotl(x1, r)
            x1 = x0 ^ x1
        x0 = x0 + ks[(i + 1) % 3]
        x1 = x1 + ks[(i + 2) % 3] + jnp.uint32(i + 1)
    return x0 ^ x1


def _bits_to_unit(bits):
    fb = (bits >> jnp.uint32(9)) | jnp.uint32(0x3F800000)
    return jax.lax.bitcast_convert_type(fb, jnp.float32) - jnp.float32(1.0)


def _bdot(a, bmat):
    return jnp.dot(a.astype(jnp.bfloat16), bmat.astype(jnp.bfloat16),
                   preferred_element_type=jnp.float32)


def _split3(x):
    hi = x.astype(jnp.bfloat16)
    r1 = x - hi.astype(jnp.float32)
    mid = r1.astype(jnp.bfloat16)
    lo = (r1 - mid.astype(jnp.float32)).astype(jnp.bfloat16)
    return hi, mid, lo


def _mega_kernel(k1_ref, b0_ref, w_ref, sth_ref, stm_ref, stl_ref,
                 noise_ref, ob_ref,
                 W1_ref, U1_ref, b1_ref, W2_ref, U2_ref, b2_ref,
                 Wm1_ref, bm1_ref, Wm2_ref, bm2_ref,
                 snh_ref, snm_ref, snl_ref, wn_ref,
                 idx_s):
    i = pl.program_id(0)
    D = DIM_STATE
    Pn = N_PARTICLES
    wv = w_ref[...]
    ks0 = k1_ref[0, 0]
    ks1 = k1_ref[0, 1]
    b0u = b0_ref[0, 0].astype(jnp.uint32)

    bb = jax.lax.broadcasted_iota(jnp.uint32, (PCHUNK, B_LOC, Pn), 1)
    pp = jax.lax.broadcasted_iota(jnp.uint32, (PCHUNK, B_LOC, Pn), 0)
    jj = jax.lax.broadcasted_iota(jnp.uint32, (PCHUNK, B_LOC, Pn), 2)
    jn = jax.lax.broadcasted_iota(jnp.int32, (PCHUNK, B_LOC, Pn), 2)
    inv = (((pp * jnp.uint32(BATCH) + bb + b0u) << jnp.uint32(10)) + jj) + ks1

    lane_b = jax.lax.broadcasted_iota(jnp.int32, (PBLK, B_LOC), 1)
    lane_j = jax.lax.broadcasted_iota(jnp.int32, (PBLK, Pn), 1)

    wr = jax.lax.rem(i, 2)
    rd = jax.lax.rem(i + 1, 2)

    @pl.when(i > 0)
    def _zero_wn():
        wn_ref[...] = jnp.zeros((PBLK, B_LOC), jnp.float32)

    def body(k, carry):
        @pl.when(jnp.logical_and(i < NPBLK, k < NCHUNK))
        def _cat():
            base_p = (i * PBLK + k * PCHUNK).astype(jnp.uint32)
            off = base_p * jnp.uint32(BATCH * N_PARTICLES)
            bits = _threefry_core(ks0, ks1, inv + off)
            f = _bits_to_unit(bits)
            u = f * (jnp.float32(1.0) - _TINY) + _TINY
            val = -jnp.log(-jnp.log(u)) + wv[None, :, :]
            m = jnp.max(val, axis=2, keepdims=True)
            cand = jnp.where(val == m, jn, jnp.int32(Pn))
            idxp = jnp.min(cand, axis=2).astype(jnp.float32)
            idx_s[pl.ds(wr * PBLK + k * PCHUNK, PCHUNK), :] = idxp

        @pl.when(jnp.logical_and(i > 0, k < B_LOC))
        def _resample_lstm():
            b = k
            idx_blk = idx_s[pl.ds(rd * PBLK, PBLK), :]
            colmask = (lane_b == b).astype(jnp.float32)
            idxb = jnp.sum(idx_blk * colmask, axis=1, keepdims=True)
            oh = (lane_j == idxb.astype(jnp.int32)).astype(jnp.bfloat16)
            g = jnp.dot(oh, sth_ref[b], preferred_element_type=jnp.float32)
            g = g + jnp.dot(oh, stm_ref[b], preferred_element_type=jnp.float32)
            g = g + jnp.dot(oh, stl_ref[b], preferred_element_type=jnp.float32)
            h1 = g[:, 0:D]
            c1 = g[:, D:2 * D]
            h2 = g[:, 2 * D:3 * D]
            c2 = g[:, 3 * D:4 * D]
            nz = noise_ref[b]
            obr = jnp.broadcast_to(ob_ref[pl.ds(b, 1), :], (PBLK, DIM_OBS))
            x = jnp.concatenate([nz, obr], axis=1)

            z1 = _bdot(x, W1_ref[...]) + _bdot(h1, U1_ref[...]) + b1_ref[...]
            i1 = jax.nn.sigmoid(z1[:, 0:D])
            f1 = jax.nn.sigmoid(z1[:, D:2 * D])
            g1 = jnp.tanh(z1[:, 2 * D:3 * D])
            o1 = jax.nn.sigmoid(z1[:, 3 * D:4 * D])
            c1n = f1 * c1 + i1 * g1
            h1n = o1 * jnp.tanh(c1n)

            z2 = _bdot(h1n, W2_ref[...]) + _bdot(h2, U2_ref[...]) + b2_ref[...]
            i2 = jax.nn.sigmoid(z2[:, 0:D])
            f2 = jax.nn.sigmoid(z2[:, D:2 * D])
            g2 = jnp.tanh(z2[:, 2 * D:3 * D])
            o2 = jax.nn.sigmoid(z2[:, 3 * D:4 * D])
            c2n = f2 * c2 + i2 * g2
            h2n = o2 * jnp.tanh(c2n)

            minp = jnp.concatenate([obr, h2n], axis=1)
            hid = jax.nn.relu(_bdot(minp, Wm1_ref[...]) + bm1_ref[...])
            wv_b = _bdot(hid, Wm2_ref[...]) + bm2_ref[0, 0]

            stn = jnp.concatenate([h1n, c1n, h2n, c2n], axis=1)
            hi, mid, lo = _split3(stn)
            snh_ref[b] = hi
            snm_ref[b] = mid
            snl_ref[b] = lo
            wn_ref[...] += wv_b * colmask

        return carry

    jax.lax.fori_loop(0, KLOOP, body, 0)


def _mega_pallas(w, sth, stm, stl, noise_t, ob, k1, b0,
                 W1, U1, b1, W2, U2, b2, Wm1, bm1, Wm2, bm2):
    D4 = 4 * DIM_STATE
    lag3 = lambda i: (0, jnp.maximum(i - 1, 0), 0)

    def full(shape):
        return pl.BlockSpec(shape, lambda i: tuple(0 for _ in shape))

    out_shapes = [
        jax.ShapeDtypeStruct((B_LOC, N_PARTICLES, D4), jnp.bfloat16),
        jax.ShapeDtypeStruct((B_LOC, N_PARTICLES, D4), jnp.bfloat16),
        jax.ShapeDtypeStruct((B_LOC, N_PARTICLES, D4), jnp.bfloat16),
        jax.ShapeDtypeStruct((N_PARTICLES, B_LOC), jnp.float32),
    ]
    out_specs = [
        pl.BlockSpec((B_LOC, PBLK, D4), lag3),
        pl.BlockSpec((B_LOC, PBLK, D4), lag3),
        pl.BlockSpec((B_LOC, PBLK, D4), lag3),
        pl.BlockSpec((PBLK, B_LOC), lambda i: (jnp.maximum(i - 1, 0), 0)),
    ]
    in_specs = [
        pl.BlockSpec(memory_space=pltpu.SMEM),
        pl.BlockSpec(memory_space=pltpu.SMEM),
        full((B_LOC, N_PARTICLES)),
        full((B_LOC, N_PARTICLES, D4)),
        full((B_LOC, N_PARTICLES, D4)),
        full((B_LOC, N_PARTICLES, D4)),
        pl.BlockSpec((B_LOC, PBLK, DIM_STATE), lag3),
        full((B_LOC, DIM_OBS)),
        full((DIM_STATE + DIM_OBS, D4)), full((DIM_STATE, D4)), full((1, D4)),
        full((DIM_STATE, D4)), full((DIM_STATE, D4)), full((1, D4)),
        full((DIM_OBS + DIM_STATE, HIDDEN)), full((1, HIDDEN)),
        full((HIDDEN, 1)), full((1, 1)),
    ]
    return pl.pallas_call(
        _mega_kernel,
        grid=(NPBLK + 1,),
        in_specs=in_specs,
        out_specs=out_specs,
        out_shape=out_shapes,
        scratch_shapes=[pltpu.VMEM((2 * PBLK, B_LOC), jnp.float32)],
    )(k1, b0, w, sth, stm, stl, noise_t, ob,
      W1, U1, b1.reshape(1, -1), W2, U2, b2.reshape(1, -1),
      Wm1, bm1.reshape(1, -1), Wm2, bm2.reshape(1, 1))


def _make_noise(k2d, b0):
    Pn, D = N_PARTICLES, DIM_STATE
    nb = jax.lax.broadcasted_iota(jnp.uint32, (B_LOC, Pn, D), 0)
    npp = jax.lax.broadcasted_iota(jnp.uint32, (B_LOC, Pn, D), 1)
    nd = jax.lax.broadcasted_iota(jnp.uint32, (B_LOC, Pn, D), 2)
    cnt = (((nb + b0.astype(jnp.uint32)) * jnp.uint32(Pn) + npp)
           << jnp.uint32(5)) + nd

    def one(k2):
        bits = _threefry_xor(k2[0], k2[1], cnt)
        f = _bits_to_unit(bits)
        nu = jnp.maximum(_LO, f * (jnp.float32(1.0) - _LO) + _LO)
        return jnp.sqrt(jnp.float32(2.0)) * jax.lax.erf_inv(nu)

    return jax.vmap(one)(k2d)


def _filter_local(b0, obs_l, k1d, k2d, W1, U1, b1, W2, U2, b2, Wm1, bm1, Wm2, bm2):
    Pn, D = N_PARTICLES, DIM_STATE
    b0_arr = b0.astype(jnp.int32).reshape(1, 1)

    sz = jnp.zeros((B_LOC, Pn, 4 * D), jnp.bfloat16)
    w = jnp.ones((B_LOC, Pn), jnp.float32) / Pn
    obs_t = jnp.transpose(obs_l, (1, 0, 2))
    noise_all = _make_noise(k2d, b0)

    def step(carry, xs):
        sth, stm, stl, w = carry
        ob, k1, noise_t = xs
        snh, snm, snl, wn_t = _mega_pallas(
            w, sth, stm, stl, noise_t, ob, k1.reshape(1, 2), b0_arr,
            W1, U1, b1, W2, U2, b2, Wm1, bm1, Wm2, bm2)
        return (snh, snm, snl, wn_t.T), None

    (sth, stm, stl, w), _ = jax.lax.scan(
        step, (sz, sz, sz, w), (obs_t, k1d, noise_all))
    st = sth.astype(jnp.float32) + stm.astype(jnp.float32) + stl.astype(jnp.float32)
    return st[..., 2 * D:3 * D], w


def _shard_filter(obs_l, k1d, k2d, W1, U1, b1, W2, U2, b2, Wm1, bm1, Wm2, bm2):
    b0 = jax.lax.axis_index("x") * B_LOC
    return _filter_local(b0, obs_l, k1d, k2d, W1, U1, b1, W2, U2, b2,
                         Wm1, bm1, Wm2, bm2)


def kernel(observations, W1, U1, b1, W2, U2, b2, Wm1, bm1, Wm2, bm2):
    T = SEQ
    keys = jax.random.split(jax.random.key(42), T)
    k12 = jax.vmap(jax.random.split)(keys)
    kd = jax.random.key_data(k12).astype(jnp.uint32)
    k1d, k2d = kd[:, 0, :], kd[:, 1, :]

    if NDEV == 1:
        return _filter_local(jnp.int32(0), observations, k1d, k2d,
                             W1, U1, b1, W2, U2, b2, Wm1, bm1, Wm2, bm2)
    mesh = jax.make_mesh((NDEV,), ("x",))
    observations = jax.reshard(
        observations, jax.NamedSharding(mesh, P("x", None, None)))
    fn = jax.shard_map(
        _shard_filter, mesh=mesh,
        in_specs=(P("x"), P(), P(), P(), P(), P(), P(), P(), P(), P(), P(), P(), P()),
        out_specs=(P("x"), P("x")),
        check_vma=False,
    )
    return fn(observations, k1d, k2d, W1, U1, b1, W2, U2, b2, Wm1, bm1, Wm2, bm2)


# speedup vs baseline: 1.2949x; 1.2949x over previous
import jax
import jax.numpy as jnp
import numpy as np
from jax.experimental import pallas as pl
from jax.experimental.pallas import tpu as pltpu
from jax.sharding import PartitionSpec as P

DIM_STATE = 32
N_PARTICLES = 1024
DIM_OBS = 32
HIDDEN = 64
BATCH = 64
SEQ = 16

NDEV = 2 if jax.device_count() >= 2 else 1
B_LOC = BATCH // NDEV
ROWS_L = B_LOC * N_PARTICLES

_TINY = np.float32(np.finfo(np.float32).tiny)
_LO = np.float32(np.nextafter(np.float32(-1.0), np.float32(0.0)))

PBLK = 128
PCHUNK = 4
NCHUNK = PBLK // PCHUNK
NPBLK = N_PARTICLES // PBLK
KLOOP = max(NCHUNK, B_LOC)


def _threefry_xor(kd0, kd1, x1):
    return _threefry_core(kd0, kd1, x1 + kd1)


def _threefry_core(ks0, ks1, x1):
    ks2 = ks0 ^ ks1 ^ jnp.uint32(0x1BD11BDA)
    x0 = jnp.zeros_like(x1) + ks0
    rots = ((13, 15, 26, 6), (17, 29, 16, 24))
    ks = (ks0, ks1, ks2)

    def rotl(x, d):
        return (x << jnp.uint32(d)) | (x >> jnp.uint32(32 - d))

    for i in range(5):
        for r in rots[i % 2]:
            x0 = x0 + x1
            x1 = rotl(x1, r)
            x1 = x0 ^ x1
        x0 = x0 + ks[(i + 1) % 3]
        x1 = x1 + ks[(i + 2) % 3] + jnp.uint32(i + 1)
    return x0 ^ x1


def _bits_to_unit(bits):
    fb = (bits >> jnp.uint32(9)) | jnp.uint32(0x3F800000)
    return jax.lax.bitcast_convert_type(fb, jnp.float32) - jnp.float32(1.0)


def _bdot(a, bmat):
    return jnp.dot(a.astype(jnp.bfloat16), bmat.astype(jnp.bfloat16),
                   preferred_element_type=jnp.float32)


def _split3(x):
    hi = x.astype(jnp.bfloat16)
    r1 = x - hi.astype(jnp.float32)
    mid = r1.astype(jnp.bfloat16)
    lo = (r1 - mid.astype(jnp.float32)).astype(jnp.bfloat16)
    return hi, mid, lo


def _mega_kernel(k1_ref, b0_ref, w_ref, sth_ref, stm_ref, stl_ref,
                 noise_ref, ob_ref,
                 W1_ref, U1_ref, b1_ref, W2_ref, U2_ref, b2_ref,
                 Wm1_ref, bm1_ref, Wm2_ref, bm2_ref,
                 snh_ref, snm_ref, snl_ref, wn_ref,
                 idx_s, stg_s):
    i = pl.program_id(0)
    D = DIM_STATE
    Pn = N_PARTICLES
    wv = w_ref[...]
    ks0 = k1_ref[0, 0]
    ks1 = k1_ref[0, 1]
    b0u = b0_ref[0, 0].astype(jnp.uint32)

    bb = jax.lax.broadcasted_iota(jnp.uint32, (PCHUNK, B_LOC, Pn), 1)
    pp = jax.lax.broadcasted_iota(jnp.uint32, (PCHUNK, B_LOC, Pn), 0)
    jj = jax.lax.broadcasted_iota(jnp.uint32, (PCHUNK, B_LOC, Pn), 2)
    jn = jax.lax.broadcasted_iota(jnp.int32, (PCHUNK, B_LOC, Pn), 2)
    inv = (((pp * jnp.uint32(BATCH) + bb + b0u) << jnp.uint32(10)) + jj) + ks1

    lane_b = jax.lax.broadcasted_iota(jnp.int32, (PBLK, B_LOC), 1)
    lane_j = jax.lax.broadcasted_iota(jnp.int32, (PBLK, Pn), 1)

    wr = jax.lax.rem(i, 2)
    rd = jax.lax.rem(i + 1, 2)

    @pl.when(i < NPBLK)
    def _cat_loop():
        def cat_body(k, carry):
            base_p = (i * PBLK + k * PCHUNK).astype(jnp.uint32)
            off = base_p * jnp.uint32(BATCH * N_PARTICLES)
            bits = _threefry_core(ks0, ks1, inv + off)
            f = _bits_to_unit(bits)
            u = f * (jnp.float32(1.0) - _TINY) + _TINY
            val = -jnp.log(-jnp.log(u)) + wv[None, :, :]
            m = jnp.max(val, axis=2, keepdims=True)
            cand = jnp.where(val == m, jn, jnp.int32(Pn))
            idxp = jnp.min(cand, axis=2).astype(jnp.float32)
            idx_s[pl.ds(wr * PBLK + k * PCHUNK, PCHUNK), :] = idxp
            return carry

        jax.lax.fori_loop(0, NCHUNK, cat_body, 0)

    @pl.when(i > 0)
    def _resample_lstm():
        idx_blk = idx_s[pl.ds(rd * PBLK, PBLK), :]

        def gather_body(b, carry):
            colmask = (lane_b == b).astype(jnp.float32)
            idxb = jnp.sum(idx_blk * colmask, axis=1, keepdims=True)
            oh = (lane_j == idxb.astype(jnp.int32)).astype(jnp.bfloat16)
            g = jnp.dot(oh, sth_ref[b], preferred_element_type=jnp.float32)
            g = g + jnp.dot(oh, stm_ref[b], preferred_element_type=jnp.float32)
            g = g + jnp.dot(oh, stl_ref[b], preferred_element_type=jnp.float32)
            stg_s[pl.ds(b * PBLK, PBLK), :] = g
            return carry

        jax.lax.fori_loop(0, B_LOC, gather_body, 0)

        R = B_LOC * PBLK
        sg = stg_s[...]
        h1 = sg[:, 0:D]
        c1 = sg[:, D:2 * D]
        h2 = sg[:, 2 * D:3 * D]
        c2 = sg[:, 3 * D:4 * D]
        nz = noise_ref[...].reshape(R, D)
        obr = jnp.broadcast_to(ob_ref[...][:, None, :],
                               (B_LOC, PBLK, DIM_OBS)).reshape(R, DIM_OBS)
        x = jnp.concatenate([nz, obr], axis=1)

        z1 = _bdot(x, W1_ref[...]) + _bdot(h1, U1_ref[...]) + b1_ref[...]
        i1 = jax.nn.sigmoid(z1[:, 0:D])
        f1 = jax.nn.sigmoid(z1[:, D:2 * D])
        g1 = jnp.tanh(z1[:, 2 * D:3 * D])
        o1 = jax.nn.sigmoid(z1[:, 3 * D:4 * D])
        c1n = f1 * c1 + i1 * g1
        h1n = o1 * jnp.tanh(c1n)

        z2 = _bdot(h1n, W2_ref[...]) + _bdot(h2, U2_ref[...]) + b2_ref[...]
        i2 = jax.nn.sigmoid(z2[:, 0:D])
        f2 = jax.nn.sigmoid(z2[:, D:2 * D])
        g2 = jnp.tanh(z2[:, 2 * D:3 * D])
        o2 = jax.nn.sigmoid(z2[:, 3 * D:4 * D])
        c2n = f2 * c2 + i2 * g2
        h2n = o2 * jnp.tanh(c2n)

        minp = jnp.concatenate([obr, h2n], axis=1)
        hid = jax.nn.relu(_bdot(minp, Wm1_ref[...]) + bm1_ref[...])
        wvn = _bdot(hid, Wm2_ref[...]) + bm2_ref[0, 0]

        stn = jnp.concatenate([h1n, c1n, h2n, c2n], axis=1)
        hi, mid, lo = _split3(stn)
        snh_ref[...] = hi.reshape(B_LOC, PBLK, 4 * D)
        snm_ref[...] = mid.reshape(B_LOC, PBLK, 4 * D)
        snl_ref[...] = lo.reshape(B_LOC, PBLK, 4 * D)
        wn_ref[...] = wvn.reshape(B_LOC, PBLK)


def _mega_pallas(w, sth, stm, stl, noise_t, ob, k1, b0,
                 W1, U1, b1, W2, U2, b2, Wm1, bm1, Wm2, bm2):
    D4 = 4 * DIM_STATE
    lag3 = lambda i: (0, jnp.maximum(i - 1, 0), 0)

    def full(shape):
        return pl.BlockSpec(shape, lambda i: tuple(0 for _ in shape))

    out_shapes = [
        jax.ShapeDtypeStruct((B_LOC, N_PARTICLES, D4), jnp.bfloat16),
        jax.ShapeDtypeStruct((B_LOC, N_PARTICLES, D4), jnp.bfloat16),
        jax.ShapeDtypeStruct((B_LOC, N_PARTICLES, D4), jnp.bfloat16),
        jax.ShapeDtypeStruct((B_LOC, N_PARTICLES), jnp.float32),
    ]
    out_specs = [
        pl.BlockSpec((B_LOC, PBLK, D4), lag3),
        pl.BlockSpec((B_LOC, PBLK, D4), lag3),
        pl.BlockSpec((B_LOC, PBLK, D4), lag3),
        pl.BlockSpec((B_LOC, PBLK), lambda i: (0, jnp.maximum(i - 1, 0))),
    ]
    in_specs = [
        pl.BlockSpec(memory_space=pltpu.SMEM),
        pl.BlockSpec(memory_space=pltpu.SMEM),
        full((B_LOC, N_PARTICLES)),
        full((B_LOC, N_PARTICLES, D4)),
        full((B_LOC, N_PARTICLES, D4)),
        full((B_LOC, N_PARTICLES, D4)),
        pl.BlockSpec((B_LOC, PBLK, DIM_STATE), lag3),
        full((B_LOC, DIM_OBS)),
        full((DIM_STATE + DIM_OBS, D4)), full((DIM_STATE, D4)), full((1, D4)),
        full((DIM_STATE, D4)), full((DIM_STATE, D4)), full((1, D4)),
        full((DIM_OBS + DIM_STATE, HIDDEN)), full((1, HIDDEN)),
        full((HIDDEN, 1)), full((1, 1)),
    ]
    return pl.pallas_call(
        _mega_kernel,
        grid=(NPBLK + 1,),
        in_specs=in_specs,
        out_specs=out_specs,
        out_shape=out_shapes,
        scratch_shapes=[pltpu.VMEM((2 * PBLK, B_LOC), jnp.float32),
                        pltpu.VMEM((B_LOC * PBLK, 4 * DIM_STATE), jnp.float32)],
    )(k1, b0, w, sth, stm, stl, noise_t, ob,
      W1, U1, b1.reshape(1, -1), W2, U2, b2.reshape(1, -1),
      Wm1, bm1.reshape(1, -1), Wm2, bm2.reshape(1, 1))


def _make_noise(k2d, b0):
    Pn, D = N_PARTICLES, DIM_STATE
    nb = jax.lax.broadcasted_iota(jnp.uint32, (B_LOC, Pn, D), 0)
    npp = jax.lax.broadcasted_iota(jnp.uint32, (B_LOC, Pn, D), 1)
    nd = jax.lax.broadcasted_iota(jnp.uint32, (B_LOC, Pn, D), 2)
    cnt = (((nb + b0.astype(jnp.uint32)) * jnp.uint32(Pn) + npp)
           << jnp.uint32(5)) + nd

    def one(k2):
        bits = _threefry_xor(k2[0], k2[1], cnt)
        f = _bits_to_unit(bits)
        nu = jnp.maximum(_LO, f * (jnp.float32(1.0) - _LO) + _LO)
        return jnp.sqrt(jnp.float32(2.0)) * jax.lax.erf_inv(nu)

    return jax.vmap(one)(k2d)


def _filter_local(b0, obs_l, k1d, k2d, W1, U1, b1, W2, U2, b2, Wm1, bm1, Wm2, bm2):
    Pn, D = N_PARTICLES, DIM_STATE
    b0_arr = b0.astype(jnp.int32).reshape(1, 1)

    sz = jnp.zeros((B_LOC, Pn, 4 * D), jnp.bfloat16)
    w = jnp.ones((B_LOC, Pn), jnp.float32) / Pn
    obs_t = jnp.transpose(obs_l, (1, 0, 2))
    noise_all = _make_noise(k2d, b0)

    def step(carry, xs):
        sth, stm, stl, w = carry
        ob, k1, noise_t = xs
        snh, snm, snl, wn = _mega_pallas(
            w, sth, stm, stl, noise_t, ob, k1.reshape(1, 2), b0_arr,
            W1, U1, b1, W2, U2, b2, Wm1, bm1, Wm2, bm2)
        return (snh, snm, snl, wn), None

    (sth, stm, stl, w), _ = jax.lax.scan(
        step, (sz, sz, sz, w), (obs_t, k1d, noise_all))
    st = sth.astype(jnp.float32) + stm.astype(jnp.float32) + stl.astype(jnp.float32)
    return st[..., 2 * D:3 * D], w


def _shard_filter(obs_l, k1d, k2d, W1, U1, b1, W2, U2, b2, Wm1, bm1, Wm2, bm2):
    b0 = jax.lax.axis_index("x") * B_LOC
    return _filter_local(b0, obs_l, k1d, k2d, W1, U1, b1, W2, U2, b2,
                         Wm1, bm1, Wm2, bm2)


def kernel(observations, W1, U1, b1, W2, U2, b2, Wm1, bm1, Wm2, bm2):
    T = SEQ
    keys = jax.random.split(jax.random.key(42), T)
    k12 = jax.vmap(jax.random.split)(keys)
    kd = jax.random.key_data(k12).astype(jnp.uint32)
    k1d, k2d = kd[:, 0, :], kd[:, 1, :]

    if NDEV == 1:
        return _filter_local(jnp.int32(0), observations, k1d, k2d,
                             W1, U1, b1, W2, U2, b2, Wm1, bm1, Wm2, bm2)
    mesh = jax.make_mesh((NDEV,), ("x",))
    observations = jax.reshard(
        observations, jax.NamedSharding(mesh, P("x", None, None)))
    fn = jax.shard_map(
        _shard_filter, mesh=mesh,
        in_specs=(P("x"), P(), P(), P(), P(), P(), P(), P(), P(), P(), P(), P(), P()),
        out_specs=(P("x"), P("x")),
        check_vma=False,
    )
    return fn(observations, k1d, k2d, W1, U1, b1, W2, U2, b2, Wm1, bm1, Wm2, bm2)


# speedup vs baseline: 1.3773x; 1.0636x over previous
import jax
import jax.numpy as jnp
import numpy as np
from jax.experimental import pallas as pl
from jax.experimental.pallas import tpu as pltpu
from jax.sharding import PartitionSpec as P

DIM_STATE = 32
N_PARTICLES = 1024
DIM_OBS = 32
HIDDEN = 64
BATCH = 64
SEQ = 16

NDEV = 2 if jax.device_count() >= 2 else 1
B_LOC = BATCH // NDEV
ROWS_L = B_LOC * N_PARTICLES

_TINY = np.float32(np.finfo(np.float32).tiny)
_LO = np.float32(np.nextafter(np.float32(-1.0), np.float32(0.0)))

PBLK = 128
PCHUNK = 8
NCHUNK = PBLK // PCHUNK
NPBLK = N_PARTICLES // PBLK
KLOOP = max(NCHUNK, B_LOC)


def _threefry_xor(kd0, kd1, x1):
    return _threefry_core(kd0, kd1, x1 + kd1)


def _threefry_core(ks0, ks1, x1):
    ks2 = ks0 ^ ks1 ^ jnp.uint32(0x1BD11BDA)
    x0 = jnp.zeros_like(x1) + ks0
    rots = ((13, 15, 26, 6), (17, 29, 16, 24))
    ks = (ks0, ks1, ks2)

    def rotl(x, d):
        return (x << jnp.uint32(d)) | (x >> jnp.uint32(32 - d))

    for i in range(5):
        for r in rots[i % 2]:
            x0 = x0 + x1
            x1 = rotl(x1, r)
            x1 = x0 ^ x1
        x0 = x0 + ks[(i + 1) % 3]
        x1 = x1 + ks[(i + 2) % 3] + jnp.uint32(i + 1)
    return x0 ^ x1


def _bits_to_unit(bits):
    fb = (bits >> jnp.uint32(9)) | jnp.uint32(0x3F800000)
    return jax.lax.bitcast_convert_type(fb, jnp.float32) - jnp.float32(1.0)


def _bdot(a, bmat):
    return jnp.dot(a.astype(jnp.bfloat16), bmat.astype(jnp.bfloat16),
                   preferred_element_type=jnp.float32)


def _split3(x):
    hi = x.astype(jnp.bfloat16)
    r1 = x - hi.astype(jnp.float32)
    mid = r1.astype(jnp.bfloat16)
    lo = (r1 - mid.astype(jnp.float32)).astype(jnp.bfloat16)
    return hi, mid, lo


def _mega_kernel(k1_ref, b0_ref, w_ref, stc_ref,
                 noise_ref, ob_ref,
                 W1_ref, U1_ref, b1_ref, W2_ref, U2_ref, b2_ref,
                 Wm1_ref, bm1_ref, Wm2_ref, bm2_ref,
                 snc_ref, wn_ref,
                 idx_s, stg_s):
    i = pl.program_id(0)
    D = DIM_STATE
    Pn = N_PARTICLES
    wv = w_ref[...]
    ks0 = k1_ref[0, 0]
    ks1 = k1_ref[0, 1]
    b0u = b0_ref[0, 0].astype(jnp.uint32)

    bb = jax.lax.broadcasted_iota(jnp.uint32, (PCHUNK, B_LOC, Pn), 1)
    pp = jax.lax.broadcasted_iota(jnp.uint32, (PCHUNK, B_LOC, Pn), 0)
    jj = jax.lax.broadcasted_iota(jnp.uint32, (PCHUNK, B_LOC, Pn), 2)
    jn = jax.lax.broadcasted_iota(jnp.int32, (PCHUNK, B_LOC, Pn), 2)
    inv = (((pp * jnp.uint32(BATCH) + bb + b0u) << jnp.uint32(10)) + jj) + ks1

    lane_b = jax.lax.broadcasted_iota(jnp.int32, (PBLK, B_LOC), 1)
    lane_j = jax.lax.broadcasted_iota(jnp.int32, (PBLK, Pn), 1)

    wr = jax.lax.rem(i, 2)
    rd = jax.lax.rem(i + 1, 2)

    @pl.when(i < NPBLK)
    def _cat_loop():
        def cat_body(k, carry):
            base_p = (i * PBLK + k * PCHUNK).astype(jnp.uint32)
            off = base_p * jnp.uint32(BATCH * N_PARTICLES)
            bits = _threefry_core(ks0, ks1, inv + off)
            f = _bits_to_unit(bits)
            u = f * (jnp.float32(1.0) - _TINY) + _TINY
            val = -jnp.log(-jnp.log(u)) + wv[None, :, :]
            m = jnp.max(val, axis=2, keepdims=True)
            cand = jnp.where(val == m, jn, jnp.int32(Pn))
            idxp = jnp.min(cand, axis=2).astype(jnp.float32)
            idx_s[pl.ds(wr * PBLK + k * PCHUNK, PCHUNK), :] = idxp
            return carry

        jax.lax.fori_loop(0, NCHUNK, cat_body, 0)

    @pl.when(i > 0)
    def _resample_lstm():
        idx_blk = idx_s[pl.ds(rd * PBLK, PBLK), :]

        def gather_body(b, carry):
            colmask = (lane_b == b).astype(jnp.float32)
            idxb = jnp.sum(idx_blk * colmask, axis=1, keepdims=True)
            oh = (lane_j == idxb.astype(jnp.int32)).astype(jnp.bfloat16)
            g3 = jnp.dot(oh, stc_ref[b], preferred_element_type=jnp.float32)
            D4 = 4 * D
            g = g3[:, 0:D4] + g3[:, D4:2 * D4] + g3[:, 2 * D4:3 * D4]
            stg_s[pl.ds(b * PBLK, PBLK), :] = g
            return carry

        jax.lax.fori_loop(0, B_LOC, gather_body, 0)

        R = B_LOC * PBLK
        sg = stg_s[...]
        h1 = sg[:, 0:D]
        c1 = sg[:, D:2 * D]
        h2 = sg[:, 2 * D:3 * D]
        c2 = sg[:, 3 * D:4 * D]
        nz = noise_ref[...].reshape(R, D)
        obr = jnp.broadcast_to(ob_ref[...][:, None, :],
                               (B_LOC, PBLK, DIM_OBS)).reshape(R, DIM_OBS)
        x = jnp.concatenate([nz, obr], axis=1)

        z1 = _bdot(x, W1_ref[...]) + _bdot(h1, U1_ref[...]) + b1_ref[...]
        i1 = jax.nn.sigmoid(z1[:, 0:D])
        f1 = jax.nn.sigmoid(z1[:, D:2 * D])
        g1 = jnp.tanh(z1[:, 2 * D:3 * D])
        o1 = jax.nn.sigmoid(z1[:, 3 * D:4 * D])
        c1n = f1 * c1 + i1 * g1
        h1n = o1 * jnp.tanh(c1n)

        z2 = _bdot(h1n, W2_ref[...]) + _bdot(h2, U2_ref[...]) + b2_ref[...]
        i2 = jax.nn.sigmoid(z2[:, 0:D])
        f2 = jax.nn.sigmoid(z2[:, D:2 * D])
        g2 = jnp.tanh(z2[:, 2 * D:3 * D])
        o2 = jax.nn.sigmoid(z2[:, 3 * D:4 * D])
        c2n = f2 * c2 + i2 * g2
        h2n = o2 * jnp.tanh(c2n)

        minp = jnp.concatenate([obr, h2n], axis=1)
        hid = jax.nn.relu(_bdot(minp, Wm1_ref[...]) + bm1_ref[...])
        wvn = _bdot(hid, Wm2_ref[...]) + bm2_ref[0, 0]

        stn = jnp.concatenate([h1n, c1n, h2n, c2n], axis=1)
        hi, mid, lo = _split3(stn)
        snc = jnp.concatenate([hi, mid, lo], axis=1)
        snc_ref[...] = snc.reshape(B_LOC, PBLK, 12 * D)
        wn_ref[...] = wvn.reshape(B_LOC, PBLK)


def _mega_pallas(w, stc, noise_t, ob, k1, b0,
                 W1, U1, b1, W2, U2, b2, Wm1, bm1, Wm2, bm2):
    D4 = 4 * DIM_STATE
    D12 = 12 * DIM_STATE
    lag3 = lambda i: (0, jnp.maximum(i - 1, 0), 0)

    def full(shape):
        return pl.BlockSpec(shape, lambda i: tuple(0 for _ in shape))

    out_shapes = [
        jax.ShapeDtypeStruct((B_LOC, N_PARTICLES, D12), jnp.bfloat16),
        jax.ShapeDtypeStruct((B_LOC, N_PARTICLES), jnp.float32),
    ]
    out_specs = [
        pl.BlockSpec((B_LOC, PBLK, D12), lag3),
        pl.BlockSpec((B_LOC, PBLK), lambda i: (0, jnp.maximum(i - 1, 0))),
    ]
    in_specs = [
        pl.BlockSpec(memory_space=pltpu.SMEM),
        pl.BlockSpec(memory_space=pltpu.SMEM),
        full((B_LOC, N_PARTICLES)),
        full((B_LOC, N_PARTICLES, D12)),
        pl.BlockSpec((B_LOC, PBLK, DIM_STATE), lag3),
        full((B_LOC, DIM_OBS)),
        full((DIM_STATE + DIM_OBS, D4)), full((DIM_STATE, D4)), full((1, D4)),
        full((DIM_STATE, D4)), full((DIM_STATE, D4)), full((1, D4)),
        full((DIM_OBS + DIM_STATE, HIDDEN)), full((1, HIDDEN)),
        full((HIDDEN, 1)), full((1, 1)),
    ]
    return pl.pallas_call(
        _mega_kernel,
        grid=(NPBLK + 1,),
        in_specs=in_specs,
        out_specs=out_specs,
        out_shape=out_shapes,
        scratch_shapes=[pltpu.VMEM((2 * PBLK, B_LOC), jnp.float32),
                        pltpu.VMEM((B_LOC * PBLK, 4 * DIM_STATE), jnp.float32)],
    )(k1, b0, w, stc, noise_t, ob,
      W1, U1, b1.reshape(1, -1), W2, U2, b2.reshape(1, -1),
      Wm1, bm1.reshape(1, -1), Wm2, bm2.reshape(1, 1))


def _make_noise(k2d, b0):
    Pn, D = N_PARTICLES, DIM_STATE
    nb = jax.lax.broadcasted_iota(jnp.uint32, (B_LOC, Pn, D), 0)
    npp = jax.lax.broadcasted_iota(jnp.uint32, (B_LOC, Pn, D), 1)
    nd = jax.lax.broadcasted_iota(jnp.uint32, (B_LOC, Pn, D), 2)
    cnt = (((nb + b0.astype(jnp.uint32)) * jnp.uint32(Pn) + npp)
           << jnp.uint32(5)) + nd

    def one(k2):
        bits = _threefry_xor(k2[0], k2[1], cnt)
        f = _bits_to_unit(bits)
        nu = jnp.maximum(_LO, f * (jnp.float32(1.0) - _LO) + _LO)
        return jnp.sqrt(jnp.float32(2.0)) * jax.lax.erf_inv(nu)

    return jax.vmap(one)(k2d)


def _filter_local(b0, obs_l, k1d, k2d, W1, U1, b1, W2, U2, b2, Wm1, bm1, Wm2, bm2):
    Pn, D = N_PARTICLES, DIM_STATE
    b0_arr = b0.astype(jnp.int32).reshape(1, 1)

    sz = jnp.zeros((B_LOC, Pn, 12 * D), jnp.bfloat16)
    w = jnp.ones((B_LOC, Pn), jnp.float32) / Pn
    obs_t = jnp.transpose(obs_l, (1, 0, 2))
    noise_all = _make_noise(k2d, b0)

    def step(carry, xs):
        stc, w = carry
        ob, k1, noise_t = xs
        snc, wn = _mega_pallas(
            w, stc, noise_t, ob, k1.reshape(1, 2), b0_arr,
            W1, U1, b1, W2, U2, b2, Wm1, bm1, Wm2, bm2)
        return (snc, wn), None

    (stc, w), _ = jax.lax.scan(step, (sz, w), (obs_t, k1d, noise_all))
    D4 = 4 * D
    st = (stc[..., 0:D4].astype(jnp.float32)
          + stc[..., D4:2 * D4].astype(jnp.float32)
          + stc[..., 2 * D4:3 * D4].astype(jnp.float32))
    return st[..., 2 * D:3 * D], w


def _shard_filter(obs_l, k1d, k2d, W1, U1, b1, W2, U2, b2, Wm1, bm1, Wm2, bm2):
    b0 = jax.lax.axis_index("x") * B_LOC
    return _filter_local(b0, obs_l, k1d, k2d, W1, U1, b1, W2, U2, b2,
                         Wm1, bm1, Wm2, bm2)


def kernel(observations, W1, U1, b1, W2, U2, b2, Wm1, bm1, Wm2, bm2):
    T = SEQ
    keys = jax.random.split(jax.random.key(42), T)
    k12 = jax.vmap(jax.random.split)(keys)
    kd = jax.random.key_data(k12).astype(jnp.uint32)
    k1d, k2d = kd[:, 0, :], kd[:, 1, :]

    if NDEV == 1:
        return _filter_local(jnp.int32(0), observations, k1d, k2d,
                             W1, U1, b1, W2, U2, b2, Wm1, bm1, Wm2, bm2)
    mesh = jax.make_mesh((NDEV,), ("x",))
    observations = jax.reshard(
        observations, jax.NamedSharding(mesh, P("x", None, None)))
    fn = jax.shard_map(
        _shard_filter, mesh=mesh,
        in_specs=(P("x"), P(), P(), P(), P(), P(), P(), P(), P(), P(), P(), P(), P()),
        out_specs=(P("x"), P("x")),
        check_vma=False,
    )
    return fn(observations, k1d, k2d, W1, U1, b1, W2, U2, b2, Wm1, bm1, Wm2, bm2)


# speedup vs baseline: 1.3880x; 1.0078x over previous
import jax
import jax.numpy as jnp
import numpy as np
from jax.experimental import pallas as pl
from jax.experimental.pallas import tpu as pltpu
from jax.sharding import PartitionSpec as P

DIM_STATE = 32
N_PARTICLES = 1024
DIM_OBS = 32
HIDDEN = 64
BATCH = 64
SEQ = 16

NDEV = 2 if jax.device_count() >= 2 else 1
B_LOC = BATCH // NDEV
ROWS_L = B_LOC * N_PARTICLES

_TINY = np.float32(np.finfo(np.float32).tiny)
_LO = np.float32(np.nextafter(np.float32(-1.0), np.float32(0.0)))

PBLK = 128
PCHUNK = 16
NCHUNK = PBLK // PCHUNK
NPBLK = N_PARTICLES // PBLK
KLOOP = max(NCHUNK, B_LOC)


def _threefry_xor(kd0, kd1, x1):
    return _threefry_core(kd0, kd1, x1 + kd1)


def _threefry_core(ks0, ks1, x1):
    ks2 = ks0 ^ ks1 ^ jnp.uint32(0x1BD11BDA)
    x0 = jnp.zeros_like(x1) + ks0
    rots = ((13, 15, 26, 6), (17, 29, 16, 24))
    ks = (ks0, ks1, ks2)

    def rotl(x, d):
        return (x << jnp.uint32(d)) | (x >> jnp.uint32(32 - d))

    for i in range(5):
        for r in rots[i % 2]:
            x0 = x0 + x1
            x1 = rotl(x1, r)
            x1 = x0 ^ x1
        x0 = x0 + ks[(i + 1) % 3]
        x1 = x1 + ks[(i + 2) % 3] + jnp.uint32(i + 1)
    return x0 ^ x1


def _bits_to_unit(bits):
    fb = (bits >> jnp.uint32(9)) | jnp.uint32(0x3F800000)
    return jax.lax.bitcast_convert_type(fb, jnp.float32) - jnp.float32(1.0)


def _bdot(a, bmat):
    return jnp.dot(a.astype(jnp.bfloat16), bmat.astype(jnp.bfloat16),
                   preferred_element_type=jnp.float32)


def _split3(x):
    hi = x.astype(jnp.bfloat16)
    r1 = x - hi.astype(jnp.float32)
    mid = r1.astype(jnp.bfloat16)
    lo = (r1 - mid.astype(jnp.float32)).astype(jnp.bfloat16)
    return hi, mid, lo


def _mega_kernel(k1_ref, b0_ref, w_ref, stc_ref,
                 noise_ref, ob_ref,
                 W1_ref, U1_ref, b1_ref, W2_ref, U2_ref, b2_ref,
                 Wm1_ref, bm1_ref, Wm2_ref, bm2_ref,
                 snc_ref, wn_ref,
                 idx_s, stg_s):
    i = pl.program_id(0)
    D = DIM_STATE
    Pn = N_PARTICLES
    wv = w_ref[...]
    ks0 = k1_ref[0, 0]
    ks1 = k1_ref[0, 1]
    b0u = b0_ref[0, 0].astype(jnp.uint32)

    bb = jax.lax.broadcasted_iota(jnp.uint32, (PCHUNK, B_LOC, Pn), 1)
    pp = jax.lax.broadcasted_iota(jnp.uint32, (PCHUNK, B_LOC, Pn), 0)
    jj = jax.lax.broadcasted_iota(jnp.uint32, (PCHUNK, B_LOC, Pn), 2)
    jn = jax.lax.broadcasted_iota(jnp.int32, (PCHUNK, B_LOC, Pn), 2)
    inv = (((pp * jnp.uint32(BATCH) + bb + b0u) << jnp.uint32(10)) + jj) + ks1

    lane_b = jax.lax.broadcasted_iota(jnp.int32, (PBLK, B_LOC), 1)
    lane_j = jax.lax.broadcasted_iota(jnp.int32, (PBLK, Pn), 1)

    wr = jax.lax.rem(i, 2)
    rd = jax.lax.rem(i + 1, 2)

    @pl.when(i < NPBLK)
    def _cat_loop():
        def cat_body(k, carry):
            base_p = (i * PBLK + k * PCHUNK).astype(jnp.uint32)
            off = base_p * jnp.uint32(BATCH * N_PARTICLES)
            bits = _threefry_core(ks0, ks1, inv + off)
            f = _bits_to_unit(bits)
            u = f * (jnp.float32(1.0) - _TINY) + _TINY
            val = -jnp.log(-jnp.log(u)) + wv[None, :, :]
            m = jnp.max(val, axis=2, keepdims=True)
            cand = jnp.where(val == m, jn, jnp.int32(Pn))
            idxp = jnp.min(cand, axis=2).astype(jnp.float32)
            idx_s[pl.ds(wr * PBLK + k * PCHUNK, PCHUNK), :] = idxp
            return carry

        jax.lax.fori_loop(0, NCHUNK, cat_body, 0)

    @pl.when(i > 0)
    def _resample_lstm():
        idx_blk = idx_s[pl.ds(rd * PBLK, PBLK), :]

        def gather_body(b, carry):
            colmask = (lane_b == b).astype(jnp.float32)
            idxb = jnp.sum(idx_blk * colmask, axis=1, keepdims=True)
            oh = (lane_j == idxb.astype(jnp.int32)).astype(jnp.bfloat16)
            g3 = jnp.dot(oh, stc_ref[b], preferred_element_type=jnp.float32)
            D4 = 4 * D
            g = g3[:, 0:D4] + g3[:, D4:2 * D4] + g3[:, 2 * D4:3 * D4]
            stg_s[pl.ds(b * PBLK, PBLK), :] = g
            return carry

        jax.lax.fori_loop(0, B_LOC, gather_body, 0)

        R = B_LOC * PBLK
        sg = stg_s[...]
        h1 = sg[:, 0:D]
        c1 = sg[:, D:2 * D]
        h2 = sg[:, 2 * D:3 * D]
        c2 = sg[:, 3 * D:4 * D]
        nz = noise_ref[...].reshape(R, D)
        obr = jnp.broadcast_to(ob_ref[...][:, None, :],
                               (B_LOC, PBLK, DIM_OBS)).reshape(R, DIM_OBS)
        x = jnp.concatenate([nz, obr], axis=1)

        z1 = _bdot(x, W1_ref[...]) + _bdot(h1, U1_ref[...]) + b1_ref[...]
        i1 = jax.nn.sigmoid(z1[:, 0:D])
        f1 = jax.nn.sigmoid(z1[:, D:2 * D])
        g1 = jnp.tanh(z1[:, 2 * D:3 * D])
        o1 = jax.nn.sigmoid(z1[:, 3 * D:4 * D])
        c1n = f1 * c1 + i1 * g1
        h1n = o1 * jnp.tanh(c1n)

        z2 = _bdot(h1n, W2_ref[...]) + _bdot(h2, U2_ref[...]) + b2_ref[...]
        i2 = jax.nn.sigmoid(z2[:, 0:D])
        f2 = jax.nn.sigmoid(z2[:, D:2 * D])
        g2 = jnp.tanh(z2[:, 2 * D:3 * D])
        o2 = jax.nn.sigmoid(z2[:, 3 * D:4 * D])
        c2n = f2 * c2 + i2 * g2
        h2n = o2 * jnp.tanh(c2n)

        minp = jnp.concatenate([obr, h2n], axis=1)
        hid = jax.nn.relu(_bdot(minp, Wm1_ref[...]) + bm1_ref[...])
        wvn = _bdot(hid, Wm2_ref[...]) + bm2_ref[0, 0]

        stn = jnp.concatenate([h1n, c1n, h2n, c2n], axis=1)
        hi, mid, lo = _split3(stn)
        snc = jnp.concatenate([hi, mid, lo], axis=1)
        snc_ref[...] = snc.reshape(B_LOC, PBLK, 12 * D)
        wn_ref[...] = wvn.reshape(B_LOC, PBLK)


def _mega_pallas(w, stc, noise_t, ob, k1, b0,
                 W1, U1, b1, W2, U2, b2, Wm1, bm1, Wm2, bm2):
    D4 = 4 * DIM_STATE
    D12 = 12 * DIM_STATE
    lag3 = lambda i: (0, jnp.maximum(i - 1, 0), 0)

    def full(shape):
        return pl.BlockSpec(shape, lambda i: tuple(0 for _ in shape))

    out_shapes = [
        jax.ShapeDtypeStruct((B_LOC, N_PARTICLES, D12), jnp.bfloat16),
        jax.ShapeDtypeStruct((B_LOC, N_PARTICLES), jnp.float32),
    ]
    out_specs = [
        pl.BlockSpec((B_LOC, PBLK, D12), lag3),
        pl.BlockSpec((B_LOC, PBLK), lambda i: (0, jnp.maximum(i - 1, 0))),
    ]
    in_specs = [
        pl.BlockSpec(memory_space=pltpu.SMEM),
        pl.BlockSpec(memory_space=pltpu.SMEM),
        full((B_LOC, N_PARTICLES)),
        full((B_LOC, N_PARTICLES, D12)),
        pl.BlockSpec((B_LOC, PBLK, DIM_STATE), lag3),
        full((B_LOC, DIM_OBS)),
        full((DIM_STATE + DIM_OBS, D4)), full((DIM_STATE, D4)), full((1, D4)),
        full((DIM_STATE, D4)), full((DIM_STATE, D4)), full((1, D4)),
        full((DIM_OBS + DIM_STATE, HIDDEN)), full((1, HIDDEN)),
        full((HIDDEN, 1)), full((1, 1)),
    ]
    return pl.pallas_call(
        _mega_kernel,
        grid=(NPBLK + 1,),
        in_specs=in_specs,
        out_specs=out_specs,
        out_shape=out_shapes,
        scratch_shapes=[pltpu.VMEM((2 * PBLK, B_LOC), jnp.float32),
                        pltpu.VMEM((B_LOC * PBLK, 4 * DIM_STATE), jnp.float32)],
    )(k1, b0, w, stc, noise_t, ob,
      W1, U1, b1.reshape(1, -1), W2, U2, b2.reshape(1, -1),
      Wm1, bm1.reshape(1, -1), Wm2, bm2.reshape(1, 1))


def _make_noise(k2d, b0):
    Pn, D = N_PARTICLES, DIM_STATE
    nb = jax.lax.broadcasted_iota(jnp.uint32, (B_LOC, Pn, D), 0)
    npp = jax.lax.broadcasted_iota(jnp.uint32, (B_LOC, Pn, D), 1)
    nd = jax.lax.broadcasted_iota(jnp.uint32, (B_LOC, Pn, D), 2)
    cnt = (((nb + b0.astype(jnp.uint32)) * jnp.uint32(Pn) + npp)
           << jnp.uint32(5)) + nd

    def one(k2):
        bits = _threefry_xor(k2[0], k2[1], cnt)
        f = _bits_to_unit(bits)
        nu = jnp.maximum(_LO, f * (jnp.float32(1.0) - _LO) + _LO)
        return jnp.sqrt(jnp.float32(2.0)) * jax.lax.erf_inv(nu)

    return jax.vmap(one)(k2d)


def _filter_local(b0, obs_l, k1d, k2d, W1, U1, b1, W2, U2, b2, Wm1, bm1, Wm2, bm2):
    Pn, D = N_PARTICLES, DIM_STATE
    b0_arr = b0.astype(jnp.int32).reshape(1, 1)

    sz = jnp.zeros((B_LOC, Pn, 12 * D), jnp.bfloat16)
    w = jnp.ones((B_LOC, Pn), jnp.float32) / Pn
    obs_t = jnp.transpose(obs_l, (1, 0, 2))
    noise_all = _make_noise(k2d, b0)

    def step(carry, xs):
        stc, w = carry
        ob, k1, noise_t = xs
        snc, wn = _mega_pallas(
            w, stc, noise_t, ob, k1.reshape(1, 2), b0_arr,
            W1, U1, b1, W2, U2, b2, Wm1, bm1, Wm2, bm2)
        return (snc, wn), None

    (stc, w), _ = jax.lax.scan(step, (sz, w), (obs_t, k1d, noise_all))
    D4 = 4 * D
    st = (stc[..., 0:D4].astype(jnp.float32)
          + stc[..., D4:2 * D4].astype(jnp.float32)
          + stc[..., 2 * D4:3 * D4].astype(jnp.float32))
    return st[..., 2 * D:3 * D], w


def _shard_filter(obs_l, k1d, k2d, W1, U1, b1, W2, U2, b2, Wm1, bm1, Wm2, bm2):
    b0 = jax.lax.axis_index("x") * B_LOC
    return _filter_local(b0, obs_l, k1d, k2d, W1, U1, b1, W2, U2, b2,
                         Wm1, bm1, Wm2, bm2)


def kernel(observations, W1, U1, b1, W2, U2, b2, Wm1, bm1, Wm2, bm2):
    T = SEQ
    keys = jax.random.split(jax.random.key(42), T)
    k12 = jax.vmap(jax.random.split)(keys)
    kd = jax.random.key_data(k12).astype(jnp.uint32)
    k1d, k2d = kd[:, 0, :], kd[:, 1, :]

    if NDEV == 1:
        return _filter_local(jnp.int32(0), observations, k1d, k2d,
                             W1, U1, b1, W2, U2, b2, Wm1, bm1, Wm2, bm2)
    mesh = jax.make_mesh((NDEV,), ("x",))
    observations = jax.reshard(
        observations, jax.NamedSharding(mesh, P("x", None, None)))
    fn = jax.shard_map(
        _shard_filter, mesh=mesh,
        in_specs=(P("x"), P(), P(), P(), P(), P(), P(), P(), P(), P(), P(), P(), P()),
        out_specs=(P("x"), P("x")),
        check_vma=False,
    )
    return fn(observations, k1d, k2d, W1, U1, b1, W2, U2, b2, Wm1, bm1, Wm2, bm2)
